# baseline (device time: 54588 ns/iter reference)
import jax
import jax.numpy as jnp
from jax import lax
from jax.experimental import pallas as pl
from jax.experimental.pallas import tpu as pltpu

N_DEV = 16
B = 16
H = 16
D = 64
BS = 16
NPG = 128
NKEY = NPG * BS
CP = 16
CKEY = CP * BS
NCH = NPG // CP
NEG = -1e30


def kernel(Q, K, V, bt, lens):
    lens2 = lens.reshape(B, 1).astype(jnp.int32)
    K4 = jnp.transpose(K, (1, 2, 3, 0))
    V4 = jnp.transpose(V, (1, 2, 3, 0))

    def body(q_ref, k_ref, v_ref, bt_ref, lens_ref, out_ref,
             macc_ref, mml_ref, cacc_ref, cml_ref,
             sa_sems, ra_sems, sm_sems, rm_sems):
        my = lax.axis_index("i")

        barrier = pltpu.get_barrier_semaphore()
        for off in range(1, N_DEV):
            pl.semaphore_signal(
                barrier, inc=1,
                device_id=((my + off) % N_DEV,),
                device_id_type=pl.DeviceIdType.MESH,
            )
        pl.semaphore_wait(barrier, N_DEV - 1)

        bt_l = bt_ref[...]
        valid = lax.broadcasted_iota(jnp.int32, (B, NPG), 1) < lens_ref[...]
        bt_v = jnp.where(valid, bt_l, -1)
        page0 = my * NPG
        p_iota = lax.broadcasted_iota(jnp.int32, (1, 1, NPG), 2)
        eq = bt_v[:, :, None] == (page0 + p_iota)
        counts = jnp.sum(eq.astype(jnp.float32), axis=1, keepdims=True)
        counts = jnp.transpose(counts, (1, 0, 2))
        maskc = counts > 0.0

        qt = jnp.transpose(q_ref[:, 0, :, :], (1, 0, 2))

        def tok(t, carry):
            m_run, l_run, acc_run = carry
            kc = k_ref[t]
            vc = v_ref[t]
            s = lax.dot_general(
                qt, kc, (((2,), (1,)), ((0,), (0,))),
                preferred_element_type=jnp.float32,
            ) * (D ** -0.5)
            s = jnp.where(maskc, s, NEG)
            m_new = jnp.maximum(m_run, jnp.max(s, axis=2, keepdims=True))
            alpha = jnp.exp(m_run - m_new)
            e = jnp.exp(s - m_new) * counts
            l_new = l_run * alpha + jnp.sum(e, axis=2, keepdims=True)
            acc_new = acc_run * alpha + lax.dot_general(
                e, vc, (((2,), (2,)), ((0,), (0,))),
                preferred_element_type=jnp.float32,
            )
            return m_new, l_new, acc_new

        m_run, l_run, acc_run = lax.fori_loop(
            0, BS, tok,
            (jnp.full((H, B, 1), NEG, dtype=jnp.float32),
             jnp.zeros((H, B, 1), dtype=jnp.float32),
             jnp.zeros((H, B, D), dtype=jnp.float32)),
        )

        macc_ref[...] = acc_run.astype(jnp.bfloat16)
        mml_ref[:, :, 0:1] = m_run
        mml_ref[:, :, 1:2] = l_run

        rdmas = []
        for off in range(1, N_DEV):
            slot = off - 1
            for src, dst, ss, rs in (
                (macc_ref, cacc_ref, sa_sems, ra_sems),
                (mml_ref, cml_ref, sm_sems, rm_sems),
            ):
                rdma = pltpu.make_async_remote_copy(
                    src_ref=src,
                    dst_ref=dst.at[slot],
                    send_sem=ss.at[slot],
                    recv_sem=rs.at[slot],
                    device_id=((my + off) % N_DEV,),
                    device_id_type=pl.DeviceIdType.MESH,
                )
                rdma.start()
                rdmas.append(rdma)

        for rdma in rdmas:
            rdma.wait_recv()

        acc_r = cacc_ref[...].astype(jnp.float32)
        ml = cml_ref[...]
        m_r = ml[:, :, :, 0:1]
        l_r = ml[:, :, :, 1:2]

        gmax = jnp.maximum(m_run, jnp.max(m_r, axis=0))
        w0 = jnp.exp(m_run - gmax)
        wr = jnp.exp(m_r - gmax[None])
        num = acc_run * w0 + jnp.sum(acc_r * wr, axis=0)
        den = l_run * w0 + jnp.sum(l_r * wr, axis=0)
        o = num / den
        out_ref[:, 0, :, :] = jnp.transpose(o, (1, 0, 2))

        for rdma in rdmas:
            rdma.wait_send()

    return pl.pallas_call(
        body,
        out_shape=jax.ShapeDtypeStruct((B, 1, H, D), jnp.float32),
        in_specs=[
            pl.BlockSpec(memory_space=pltpu.VMEM),
            pl.BlockSpec(memory_space=pltpu.VMEM),
            pl.BlockSpec(memory_space=pltpu.VMEM),
            pl.BlockSpec(memory_space=pltpu.VMEM),
            pl.BlockSpec(memory_space=pltpu.VMEM),
        ],
        out_specs=pl.BlockSpec(memory_space=pltpu.VMEM),
        scratch_shapes=[
            pltpu.VMEM((H, B, D), jnp.bfloat16),
            pltpu.VMEM((H, B, 2), jnp.float32),
            pltpu.VMEM((N_DEV - 1, H, B, D), jnp.bfloat16),
            pltpu.VMEM((N_DEV - 1, H, B, 2), jnp.float32),
            pltpu.SemaphoreType.DMA((N_DEV - 1,)),
            pltpu.SemaphoreType.DMA((N_DEV - 1,)),
            pltpu.SemaphoreType.DMA((N_DEV - 1,)),
            pltpu.SemaphoreType.DMA((N_DEV - 1,)),
        ],
        compiler_params=pltpu.CompilerParams(collective_id=0),
    )(Q, K4, V4, bt, lens2)


# device time: 35751 ns/iter; 1.5269x vs baseline; 1.5269x over previous
import jax
import jax.numpy as jnp
from jax import lax
from jax.experimental import pallas as pl
from jax.experimental.pallas import tpu as pltpu

N_DEV = 16
B = 16
H = 16
D = 64
BS = 16
NPG = 128
NKEY = NPG * BS
CP = 16
CKEY = CP * BS
NCH = NPG // CP
NEG = -1e30


def kernel(Q, K, V, bt, lens):
    lens2 = lens.reshape(B, 1).astype(jnp.int32)
    K4 = jnp.transpose(K, (1, 2, 3, 0))
    V4 = jnp.transpose(V, (1, 2, 3, 0))

    def body(q_ref, k_ref, v_ref, bt_ref, lens_ref, out_ref,
             mine_ref, comm_ref, send_sems, recv_sems):
        my = lax.axis_index("i")

        barrier = pltpu.get_barrier_semaphore()
        for off in range(1, N_DEV):
            pl.semaphore_signal(
                barrier, inc=1,
                device_id=((my + off) % N_DEV,),
                device_id_type=pl.DeviceIdType.MESH,
            )
        pl.semaphore_wait(barrier, N_DEV - 1)

        bt_l = bt_ref[...]
        valid = lax.broadcasted_iota(jnp.int32, (B, NPG), 1) < lens_ref[...]
        bt_v = jnp.where(valid, bt_l, -1)
        page0 = my * NPG
        p_iota = lax.broadcasted_iota(jnp.int32, (1, 1, NPG), 2)
        eq = bt_v[:, :, None] == (page0 + p_iota)
        counts = jnp.sum(eq.astype(jnp.float32), axis=1, keepdims=True)
        counts = jnp.transpose(counts, (1, 0, 2))
        maskc = counts > 0.0

        qt = jnp.transpose(q_ref[:, 0, :, :], (1, 0, 2))

        def tok(t, carry):
            m_run, l_run, acc_run = carry
            kc = k_ref[t]
            vc = v_ref[t]
            s = lax.dot_general(
                qt, kc, (((2,), (1,)), ((0,), (0,))),
                preferred_element_type=jnp.float32,
            ) * (D ** -0.5)
            s = jnp.where(maskc, s, NEG)
            m_new = jnp.maximum(m_run, jnp.max(s, axis=2, keepdims=True))
            alpha = jnp.exp(m_run - m_new)
            e = jnp.exp(s - m_new) * counts
            l_new = l_run * alpha + jnp.sum(e, axis=2, keepdims=True)
            acc_new = acc_run * alpha + lax.dot_general(
                e, vc, (((2,), (2,)), ((0,), (0,))),
                preferred_element_type=jnp.float32,
            )
            return m_new, l_new, acc_new

        m_run, l_run, acc_run = lax.fori_loop(
            0, BS, tok,
            (jnp.full((H, B, 1), NEG, dtype=jnp.float32),
             jnp.zeros((H, B, 1), dtype=jnp.float32),
             jnp.zeros((H, B, D), dtype=jnp.float32)),
        )

        mine_ref[:, :, 0:D] = acc_run.astype(jnp.bfloat16)
        m_hi = m_run.astype(jnp.bfloat16)
        l_hi = l_run.astype(jnp.bfloat16)
        mine_ref[:, :, D:D + 1] = m_hi
        mine_ref[:, :, D + 1:D + 2] = (
            m_run - m_hi.astype(jnp.float32)).astype(jnp.bfloat16)
        mine_ref[:, :, D + 2:D + 3] = l_hi
        mine_ref[:, :, D + 3:D + 4] = (
            l_run - l_hi.astype(jnp.float32)).astype(jnp.bfloat16)

        rdmas = []
        for off in range(1, N_DEV):
            slot = off - 1
            rdma = pltpu.make_async_remote_copy(
                src_ref=mine_ref,
                dst_ref=comm_ref.at[slot],
                send_sem=send_sems.at[slot],
                recv_sem=recv_sems.at[slot],
                device_id=((my + off) % N_DEV,),
                device_id_type=pl.DeviceIdType.MESH,
            )
            rdma.start()
            rdmas.append(rdma)

        for rdma in rdmas:
            rdma.wait_recv()

        allc = comm_ref[...]
        acc_r = allc[:, :, :, 0:D].astype(jnp.float32)
        m_r = (allc[:, :, :, D:D + 1].astype(jnp.float32)
               + allc[:, :, :, D + 1:D + 2].astype(jnp.float32))
        l_r = (allc[:, :, :, D + 2:D + 3].astype(jnp.float32)
               + allc[:, :, :, D + 3:D + 4].astype(jnp.float32))

        gmax = jnp.maximum(m_run, jnp.max(m_r, axis=0))
        w0 = jnp.exp(m_run - gmax)
        wr = jnp.exp(m_r - gmax[None])
        num = acc_run * w0 + jnp.sum(acc_r * wr, axis=0)
        den = l_run * w0 + jnp.sum(l_r * wr, axis=0)
        o = num / den
        out_ref[:, 0, :, :] = jnp.transpose(o, (1, 0, 2))

        for rdma in rdmas:
            rdma.wait_send()

    return pl.pallas_call(
        body,
        out_shape=jax.ShapeDtypeStruct((B, 1, H, D), jnp.float32),
        in_specs=[
            pl.BlockSpec(memory_space=pltpu.VMEM),
            pl.BlockSpec(memory_space=pltpu.VMEM),
            pl.BlockSpec(memory_space=pltpu.VMEM),
            pl.BlockSpec(memory_space=pltpu.VMEM),
            pl.BlockSpec(memory_space=pltpu.VMEM),
        ],
        out_specs=pl.BlockSpec(memory_space=pltpu.VMEM),
        scratch_shapes=[
            pltpu.VMEM((H, B, D + 4), jnp.bfloat16),
            pltpu.VMEM((N_DEV - 1, H, B, D + 4), jnp.bfloat16),
            pltpu.SemaphoreType.DMA((N_DEV - 1,)),
            pltpu.SemaphoreType.DMA((N_DEV - 1,)),
        ],
        compiler_params=pltpu.CompilerParams(collective_id=0),
    )(Q, K4, V4, bt, lens2)


# device time: 31714 ns/iter; 1.7213x vs baseline; 1.1273x over previous
import jax
import jax.numpy as jnp
from jax import lax
from jax.experimental import pallas as pl
from jax.experimental.pallas import tpu as pltpu

N_DEV = 16
B = 16
H = 16
D = 64
BS = 16
NPG = 128
NEG = -1e30
BF = jnp.bfloat16


def _pack(buf_ref, m, l, acc):
    buf_ref[:, :, 0:D] = acc.astype(BF)
    m_hi = m.astype(BF)
    l_hi = l.astype(BF)
    buf_ref[:, :, D:D + 1] = m_hi
    buf_ref[:, :, D + 1:D + 2] = (m - m_hi.astype(jnp.float32)).astype(BF)
    buf_ref[:, :, D + 2:D + 3] = l_hi
    buf_ref[:, :, D + 3:D + 4] = (l - l_hi.astype(jnp.float32)).astype(BF)


def _merge(m, l, acc, comm_ref):
    allc = comm_ref[...]
    acc_r = allc[:, :, :, 0:D].astype(jnp.float32)
    m_r = (allc[:, :, :, D:D + 1].astype(jnp.float32)
           + allc[:, :, :, D + 1:D + 2].astype(jnp.float32))
    l_r = (allc[:, :, :, D + 2:D + 3].astype(jnp.float32)
           + allc[:, :, :, D + 3:D + 4].astype(jnp.float32))
    gmax = jnp.maximum(m, jnp.max(m_r, axis=0))
    w0 = jnp.exp(m - gmax)
    wr = jnp.exp(m_r - gmax[None])
    num = acc * w0 + jnp.sum(acc_r * wr, axis=0)
    den = l * w0 + jnp.sum(l_r * wr, axis=0)
    return gmax, den, num


def kernel(Q, K, V, bt, lens):
    lens2 = lens.reshape(B, 1).astype(jnp.int32)
    K4 = jnp.transpose(K, (1, 2, 3, 0))
    V4 = jnp.transpose(V, (1, 2, 3, 0))

    def body(q_ref, k_ref, v_ref, bt_ref, lens_ref, out_ref,
             mine1_ref, mine2_ref, comm1_ref, comm2_ref,
             s1s, s1r, s2s, s2r):
        my = lax.axis_index("i")
        base = (my // 4) * 4
        qpos = my % 4
        zpos = my // 4
        col = my % 4

        def plane_peer(j):
            return base + (qpos + j) % 4

        def col_peer(j):
            return col + 4 * ((zpos + j) % 4)

        barrier = pltpu.get_barrier_semaphore()
        for j in range(1, 4):
            for peer in (plane_peer(j), col_peer(j)):
                pl.semaphore_signal(
                    barrier, inc=1, device_id=(peer,),
                    device_id_type=pl.DeviceIdType.MESH,
                )
        pl.semaphore_wait(barrier, 6)

        bt_l = bt_ref[...]
        valid = lax.broadcasted_iota(jnp.int32, (B, NPG), 1) < lens_ref[...]
        bt_v = jnp.where(valid, bt_l, -1)
        page0 = my * NPG
        p_iota = lax.broadcasted_iota(jnp.int32, (1, 1, NPG), 2)
        eq = bt_v[:, :, None] == (page0 + p_iota)
        counts = jnp.sum(eq.astype(jnp.float32), axis=1, keepdims=True)
        counts = jnp.transpose(counts, (1, 0, 2))
        maskc = counts > 0.0

        qt = jnp.transpose(q_ref[:, 0, :, :], (1, 0, 2))

        def tok(t, carry):
            m_run, l_run, acc_run = carry
            kc = k_ref[t]
            vc = v_ref[t]
            s = lax.dot_general(
                qt, kc, (((2,), (1,)), ((0,), (0,))),
                preferred_element_type=jnp.float32,
            ) * (D ** -0.5)
            s = jnp.where(maskc, s, NEG)
            m_new = jnp.maximum(m_run, jnp.max(s, axis=2, keepdims=True))
            alpha = jnp.exp(m_run - m_new)
            e = jnp.exp(s - m_new) * counts
            l_new = l_run * alpha + jnp.sum(e, axis=2, keepdims=True)
            acc_new = acc_run * alpha + lax.dot_general(
                e, vc, (((2,), (2,)), ((0,), (0,))),
                preferred_element_type=jnp.float32,
            )
            return m_new, l_new, acc_new

        m1, l1, acc1 = lax.fori_loop(
            0, BS, tok,
            (jnp.full((H, B, 1), NEG, dtype=jnp.float32),
             jnp.zeros((H, B, 1), dtype=jnp.float32),
             jnp.zeros((H, B, D), dtype=jnp.float32)),
        )

        _pack(mine1_ref, m1, l1, acc1)
        r1 = []
        for j in range(1, 4):
            rdma = pltpu.make_async_remote_copy(
                src_ref=mine1_ref,
                dst_ref=comm1_ref.at[j - 1],
                send_sem=s1s.at[j - 1],
                recv_sem=s1r.at[j - 1],
                device_id=(plane_peer(j),),
                device_id_type=pl.DeviceIdType.MESH,
            )
            rdma.start()
            r1.append(rdma)
        for rdma in r1:
            rdma.wait_recv()
        m2, l2, acc2 = _merge(m1, l1, acc1, comm1_ref)

        _pack(mine2_ref, m2, l2, acc2)
        r2 = []
        for j in range(1, 4):
            rdma = pltpu.make_async_remote_copy(
                src_ref=mine2_ref,
                dst_ref=comm2_ref.at[j - 1],
                send_sem=s2s.at[j - 1],
                recv_sem=s2r.at[j - 1],
                device_id=(col_peer(j),),
                device_id_type=pl.DeviceIdType.MESH,
            )
            rdma.start()
            r2.append(rdma)
        for rdma in r2:
            rdma.wait_recv()
        gmax, den, num = _merge(m2, l2, acc2, comm2_ref)

        o = num / den
        out_ref[:, 0, :, :] = jnp.transpose(o, (1, 0, 2))

        for rdma in r1 + r2:
            rdma.wait_send()

    return pl.pallas_call(
        body,
        out_shape=jax.ShapeDtypeStruct((B, 1, H, D), jnp.float32),
        in_specs=[pl.BlockSpec(memory_space=pltpu.VMEM)] * 5,
        out_specs=pl.BlockSpec(memory_space=pltpu.VMEM),
        scratch_shapes=[
            pltpu.VMEM((H, B, D + 4), BF),
            pltpu.VMEM((H, B, D + 4), BF),
            pltpu.VMEM((3, H, B, D + 4), BF),
            pltpu.VMEM((3, H, B, D + 4), BF),
            pltpu.SemaphoreType.DMA((3,)),
            pltpu.SemaphoreType.DMA((3,)),
            pltpu.SemaphoreType.DMA((3,)),
            pltpu.SemaphoreType.DMA((3,)),
        ],
        compiler_params=pltpu.CompilerParams(collective_id=0),
    )(Q, K4, V4, bt, lens2)
